# baseline (speedup 1.0000x reference)
import jax
import jax.numpy as jnp
from jax.experimental import pallas as pl
from jax.experimental.pallas import tpu as pltpu

B, L, D, H = 128, 64, 256, 128
NUM_CLASSES, HASH_BITS = 100, 48


def _dranet_kernel(seq_ref, sl_col_ref, sl_row_ref,
                   Wcomb_ref, Whh_t_ref, bcomb_ref, b_hh_ref,
                   Wq_t_ref, Wp_t_ref, bp_ref, Wh_t_ref, bh_ref,
                   pred_ref, hash_ref, att_ref,
                   h_ref, k_ref, v_ref):
    t = pl.program_id(0)

    @pl.when(t == 0)
    def _init():
        h_ref[...] = jnp.zeros((B, H), jnp.float32)

    x = seq_ref[...]
    acts = jnp.dot(x, Wcomb_ref[...],
                   preferred_element_type=jnp.float32)
    gi = acts[:, :3 * H] + bcomb_ref[...]
    v_ref[pl.ds(t, 1)] = jnp.maximum(acts[:, 3 * H:4 * H], 0.0)[None]
    k_ref[pl.ds(t, 1)] = acts[:, 4 * H:][None]

    h = h_ref[...]
    gh = jnp.dot(h, Whh_t_ref[...], preferred_element_type=jnp.float32)
    rz = jax.nn.sigmoid(gi[:, :2 * H] + gh[:, :2 * H])
    r = rz[:, :H]
    z = rz[:, H:]
    n = jnp.tanh(gi[:, 2 * H:] + r * (gh[:, 2 * H:] + b_hh_ref[:, 2 * H:]))
    h_new = (1.0 - z) * n + z * h
    sl_col = sl_col_ref[...]
    hn = jnp.where(t < sl_col, h_new, h)
    h_ref[...] = hn

    @pl.when(t == L - 1)
    def _tail():
        query = jnp.dot(hn, Wq_t_ref[...],
                        preferred_element_type=jnp.float32)
        dist = jnp.sum(k_ref[...] * query[None, :, :], axis=2)
        s = dist * (1.0 / jnp.sqrt(jnp.float32(H)))
        m = jnp.max(s, axis=0, keepdims=True)
        e = jnp.exp(s - m)
        pos_l = jax.lax.broadcasted_iota(jnp.int32, (L, B), 0)
        sl_row = sl_row_ref[...]
        e = jnp.where(pos_l < sl_row, e, 0.0)
        att = e / jnp.sum(e, axis=0, keepdims=True)

        out = jnp.sum(att[:, :, None] * v_ref[...], axis=0) + query

        iota_j = jax.lax.broadcasted_iota(jnp.int32, (B, B), 0)
        iota_i = jax.lax.broadcasted_iota(jnp.int32, (B, B), 1)
        before = (sl_col > sl_row) | ((sl_col == sl_row) & (iota_j < iota_i))
        rank_row = jnp.sum(before.astype(jnp.int32), axis=0, keepdims=True)
        perm = (iota_j == rank_row).astype(jnp.float32)
        att_ref[...] = jnp.dot(perm, att.T, preferred_element_type=jnp.float32)

        pred_ref[...] = jnp.dot(out, Wp_t_ref[...],
                                preferred_element_type=jnp.float32) + bp_ref[...]
        hash_ref[...] = jnp.tanh(jnp.dot(out, Wh_t_ref[...],
                                         preferred_element_type=jnp.float32)
                                 + bh_ref[...])


@jax.jit
def kernel(sequence, sq_len, W_ih, W_hh, b_ih, b_hh, Wq, Wk, Wv, Wp, bp, Wh, bh):
    W_comb = jnp.concatenate([W_ih.T, Wv.T, Wk.T], axis=1)
    b_comb = b_ih + jnp.concatenate(
        [b_hh[:2 * H], jnp.zeros((H,), jnp.float32)])

    def c2(shape):
        return pl.BlockSpec(shape, lambda t: (0, 0))

    predict, hash_code, att_sq = pl.pallas_call(
        _dranet_kernel,
        grid=(L,),
        in_specs=[
            pl.BlockSpec((B, D), lambda t: (0, t)),
            c2((B, 1)), c2((1, B)),
            c2((D, 5 * H)), c2((H, 3 * H)), c2((1, 3 * H)), c2((1, 3 * H)),
            c2((H, H)), c2((H, NUM_CLASSES)), c2((1, NUM_CLASSES)),
            c2((H, HASH_BITS)), c2((1, HASH_BITS)),
        ],
        out_specs=[
            c2((B, NUM_CLASSES)),
            c2((B, HASH_BITS)),
            c2((B, L)),
        ],
        out_shape=[
            jax.ShapeDtypeStruct((B, NUM_CLASSES), jnp.float32),
            jax.ShapeDtypeStruct((B, HASH_BITS), jnp.float32),
            jax.ShapeDtypeStruct((B, L), jnp.float32),
        ],
        scratch_shapes=[
            pltpu.VMEM((B, H), jnp.float32),
            pltpu.VMEM((L, B, H), jnp.float32),
            pltpu.VMEM((L, B, H), jnp.float32),
        ],
        compiler_params=pltpu.CompilerParams(
            vmem_limit_bytes=100 * 1024 * 1024,
        ),
    )(sequence.reshape(B, L * D),
      sq_len.reshape(B, 1),
      sq_len.reshape(1, B),
      W_comb, W_hh.T,
      b_comb.reshape(1, -1), b_hh.reshape(1, -1),
      Wq.T, Wp.T, bp.reshape(1, -1),
      Wh.T, bh.reshape(1, -1))
    return predict, hash_code, att_sq


# speedup vs baseline: 1.5337x; 1.5337x over previous
import jax
import jax.numpy as jnp
from jax.experimental import pallas as pl
from jax.experimental.pallas import tpu as pltpu

B, L, D, H = 128, 64, 256, 128
NUM_CLASSES, HASH_BITS = 100, 48


T = 8


def _dranet_kernel(seq_ref, sl_col_ref, sl_row_ref,
                   Wcomb_ref, Whh_t_ref, bcomb_ref, b_hh_ref,
                   Wq_t_ref, Wp_t_ref, bp_ref, Wh_t_ref, bh_ref,
                   pred_ref, hash_ref, att_ref,
                   h_ref, k_ref, v_ref):
    tb = pl.program_id(0)

    @pl.when(tb == 0)
    def _init():
        h_ref[...] = jnp.zeros((B, H), jnp.float32)

    xs = seq_ref[...]
    sl_col = sl_col_ref[...]
    Whh_t = Whh_t_ref[...]
    bcomb = bcomb_ref[...]
    b_hh_n = b_hh_ref[:, 2 * H:]
    hn = h_ref[...]
    for u in range(T):
        t = tb * T + u
        x = xs[:, u * D:(u + 1) * D]
        acts = jnp.dot(x, Wcomb_ref[...],
                       preferred_element_type=jnp.float32)
        gi = acts[:, :3 * H] + bcomb
        v_ref[pl.ds(t, 1)] = jnp.maximum(acts[:, 3 * H:4 * H], 0.0)[None]
        k_ref[pl.ds(t, 1)] = acts[:, 4 * H:][None]

        h = hn
        gh = jnp.dot(h, Whh_t, preferred_element_type=jnp.float32)
        rz = jax.nn.sigmoid(gi[:, :2 * H] + gh[:, :2 * H])
        r = rz[:, :H]
        z = rz[:, H:]
        n = jnp.tanh(gi[:, 2 * H:] + r * (gh[:, 2 * H:] + b_hh_n))
        h_new = (1.0 - z) * n + z * h
        hn = jnp.where(t < sl_col, h_new, h)
    h_ref[...] = hn

    @pl.when(tb == L // T - 1)
    def _tail():
        query = jnp.dot(hn, Wq_t_ref[...],
                        preferred_element_type=jnp.float32)
        dist = jnp.sum(k_ref[...] * query[None, :, :], axis=2)
        s = dist * (1.0 / jnp.sqrt(jnp.float32(H)))
        m = jnp.max(s, axis=0, keepdims=True)
        e = jnp.exp(s - m)
        pos_l = jax.lax.broadcasted_iota(jnp.int32, (L, B), 0)
        sl_row = sl_row_ref[...]
        e = jnp.where(pos_l < sl_row, e, 0.0)
        att = e / jnp.sum(e, axis=0, keepdims=True)

        out = jnp.sum(att[:, :, None] * v_ref[...], axis=0) + query

        iota_j = jax.lax.broadcasted_iota(jnp.int32, (B, B), 0)
        iota_i = jax.lax.broadcasted_iota(jnp.int32, (B, B), 1)
        before = (sl_col > sl_row) | ((sl_col == sl_row) & (iota_j < iota_i))
        rank_row = jnp.sum(before.astype(jnp.int32), axis=0, keepdims=True)
        perm = (iota_j == rank_row).astype(jnp.float32)
        att_ref[...] = jnp.dot(perm, att.T, preferred_element_type=jnp.float32)

        pred_ref[...] = jnp.dot(out, Wp_t_ref[...],
                                preferred_element_type=jnp.float32) + bp_ref[...]
        hash_ref[...] = jnp.tanh(jnp.dot(out, Wh_t_ref[...],
                                         preferred_element_type=jnp.float32)
                                 + bh_ref[...])


@jax.jit
def kernel(sequence, sq_len, W_ih, W_hh, b_ih, b_hh, Wq, Wk, Wv, Wp, bp, Wh, bh):
    W_comb = jnp.concatenate([W_ih.T, Wv.T, Wk.T], axis=1)
    b_comb = b_ih + jnp.concatenate(
        [b_hh[:2 * H], jnp.zeros((H,), jnp.float32)])

    def c2(shape):
        return pl.BlockSpec(shape, lambda t: (0, 0))

    predict, hash_code, att_sq = pl.pallas_call(
        _dranet_kernel,
        grid=(L // T,),
        in_specs=[
            pl.BlockSpec((B, T * D), lambda t: (0, t)),
            c2((B, 1)), c2((1, B)),
            c2((D, 5 * H)), c2((H, 3 * H)), c2((1, 3 * H)), c2((1, 3 * H)),
            c2((H, H)), c2((H, NUM_CLASSES)), c2((1, NUM_CLASSES)),
            c2((H, HASH_BITS)), c2((1, HASH_BITS)),
        ],
        out_specs=[
            c2((B, NUM_CLASSES)),
            c2((B, HASH_BITS)),
            c2((B, L)),
        ],
        out_shape=[
            jax.ShapeDtypeStruct((B, NUM_CLASSES), jnp.float32),
            jax.ShapeDtypeStruct((B, HASH_BITS), jnp.float32),
            jax.ShapeDtypeStruct((B, L), jnp.float32),
        ],
        scratch_shapes=[
            pltpu.VMEM((B, H), jnp.float32),
            pltpu.VMEM((L, B, H), jnp.float32),
            pltpu.VMEM((L, B, H), jnp.float32),
        ],
        compiler_params=pltpu.CompilerParams(
            vmem_limit_bytes=100 * 1024 * 1024,
        ),
    )(sequence.reshape(B, L * D),
      sq_len.reshape(B, 1),
      sq_len.reshape(1, B),
      W_comb, W_hh.T,
      b_comb.reshape(1, -1), b_hh.reshape(1, -1),
      Wq.T, Wp.T, bp.reshape(1, -1),
      Wh.T, bh.reshape(1, -1))
    return predict, hash_code, att_sq
